# baseline (speedup 1.0000x reference)
import jax
import jax.numpy as jnp
from jax.experimental import pallas as pl
from jax.experimental.pallas import tpu as pltpu


def _copy_body(x_ref, o_ref):
    o_ref[...] = x_ref[...]


def kernel(x, active, ln_weight, ln_bias):
    B, C, H, W = x.shape
    out = pl.pallas_call(
        _copy_body,
        grid=(B,),
        in_specs=[pl.BlockSpec((1, C, H, W), lambda i: (i, 0, 0, 0))],
        out_specs=pl.BlockSpec((1, C, H, W), lambda i: (i, 0, 0, 0)),
        out_shape=jax.ShapeDtypeStruct((B, C, H, W), jnp.float32),
        compiler_params=pltpu.CompilerParams(
            dimension_semantics=("parallel",),
        ),
    )(x)
    return out


# speedup vs baseline: 9.3187x; 9.3187x over previous
import jax
import jax.numpy as jnp
from jax.experimental import pallas as pl
from jax.experimental.pallas import tpu as pltpu

_EPS = 1e-6


def _ln_body(x_ref, m_ref, w_ref, b_ref, o_ref):
    xb = x_ref[0]
    C = xb.shape[1]
    s1 = jnp.sum(xb, axis=1, keepdims=True)
    s2 = jnp.sum(xb * xb, axis=1, keepdims=True)
    mean = s1 * (1.0 / C)
    var = s2 * (1.0 / C) - mean * mean
    mask = m_ref[0]
    s = jax.lax.rsqrt(var + _EPS) * mask
    o_ref[0] = ((xb - mean) * s) * w_ref[...] + b_ref[...] * mask


def kernel(x, active, ln_weight, ln_bias):
    B, C, H, W = x.shape
    P = H * W
    sh = H // active.shape[2]
    sw = W // active.shape[3]
    a = active[:, 0].astype(jnp.float32)
    mask = jnp.repeat(jnp.repeat(a, sh, axis=1), sw, axis=2)
    mask = (mask != 0.0).astype(jnp.float32).reshape(B, P, 1)

    xt = jnp.transpose(x, (0, 2, 3, 1)).reshape(B, P, C)
    w2 = ln_weight.reshape(1, C)
    b2 = ln_bias.reshape(1, C)

    out = pl.pallas_call(
        _ln_body,
        grid=(B,),
        in_specs=[
            pl.BlockSpec((1, P, C), lambda i: (i, 0, 0)),
            pl.BlockSpec((1, P, 1), lambda i: (i, 0, 0)),
            pl.BlockSpec((1, C), lambda i: (0, 0)),
            pl.BlockSpec((1, C), lambda i: (0, 0)),
        ],
        out_specs=pl.BlockSpec((1, P, C), lambda i: (i, 0, 0)),
        out_shape=jax.ShapeDtypeStruct((B, P, C), jnp.float32),
        compiler_params=pltpu.CompilerParams(
            dimension_semantics=("parallel",),
        ),
    )(xt, mask, w2, b2)
    return jnp.transpose(out.reshape(B, H, W, C), (0, 3, 1, 2))
